# baseline (speedup 1.0000x reference)
import functools

import jax
import jax.numpy as jnp
from jax import lax
from jax.experimental import pallas as pl
from jax.experimental.pallas import tpu as pltpu
from jax.experimental.pallas import tpu_sc as plsc

_CH = 128
_NCORE = 2
_NSUB = 16
_NWORK = _NCORE * _NSUB


def _pad_up(v, m):
    return (v + m - 1) // m * m


def _make_segsum(n_pad, w, nch, with_cnt):
    rpt = n_pad // _NSUB
    mesh = plsc.VectorSubcoreMesh(core_axis_name="c", subcore_axis_name="s")

    grp = 5 if with_cnt else 10
    assert nch % (2 * grp) == 0
    ngrp = nch // grp

    out_shape = jax.ShapeDtypeStruct((_NCORE, n_pad, w), jnp.float32)
    scratch = [
        pltpu.VMEM((nch, _CH), jnp.int32),
        pltpu.VMEM((nch, _CH), jnp.int32),
        *[pltpu.VMEM((_CH, w), jnp.float32) for _ in range(2 * grp)],
        pltpu.VMEM_SHARED((n_pad, w), jnp.float32),
        pltpu.SemaphoreType.DMA,
        pltpu.SemaphoreType.DMA,
    ]
    if with_cnt:
        scratch += [
            pltpu.VMEM((_CH, w), jnp.float32),
            pltpu.VMEM_SHARED((n_pad, w), jnp.float32),
            pltpu.SemaphoreType.DMA,
        ]

    @functools.partial(
        pl.kernel,
        out_type=[out_shape, out_shape] if with_cnt else out_shape,
        mesh=mesh,
        compiler_params=pltpu.CompilerParams(use_tc_tiling_on_sc=False),
        scratch_types=scratch,
    )
    def seg(table, src2d, dst2d, *rest):
        if with_cnt:
            out, cout_hbm = rest[0], rest[1]
            rest = rest[2:]
        else:
            out = rest[0]
            rest = rest[1:]
        srcv, dstv = rest[0], rest[1]
        bufs = (rest[2:2 + grp], rest[2 + grp:2 + 2 * grp])
        acc = rest[2 + 2 * grp]
        gsem, ssem = rest[3 + 2 * grp], rest[4 + 2 * grp]
        if with_cnt:
            obuf, cacc, osem = (rest[5 + 2 * grp], rest[6 + 2 * grp],
                                rest[7 + 2 * grp])
        cid = lax.axis_index("c")
        sid = lax.axis_index("s")
        wid = cid * _NSUB + sid

        pltpu.sync_copy(src2d.at[pl.ds(wid * nch, nch)], srcv)
        pltpu.sync_copy(dst2d.at[pl.ds(wid * nch, nch)], dstv)

        zv = jnp.zeros((16,), jnp.float32)

        def zrow(i, carry):
            for j in range(w // 16):
                bufs[0][0][i, pl.ds(j * 16, 16)] = zv
            return carry

        lax.fori_loop(0, _CH, zrow, 0)

        def zcopy(k, carry):
            pltpu.sync_copy(bufs[0][0],
                            acc.at[pl.ds(sid * rpt + k * _CH, _CH)])
            if with_cnt:
                pltpu.sync_copy(bufs[0][0],
                                cacc.at[pl.ds(sid * rpt + k * _CH, _CH)])
            return carry

        lax.fori_loop(0, rpt // _CH, zcopy, 0)

        if with_cnt:
            ov = jnp.ones((16,), jnp.float32)

            def orow(i, carry):
                for j in range(w // 16):
                    obuf[i, pl.ds(j * 16, 16)] = ov
                return carry

            lax.fori_loop(0, _CH, orow, 0)

        plsc.subcore_barrier()

        npair = ngrp // 2

        def wait_gathers(bset):
            for j in range(grp):
                pltpu.make_async_copy(table.at[srcv.at[0]], bset[j],
                                      gsem).wait()

        def fire_gathers(bset, base):
            for j in range(grp):
                pltpu.async_copy(table.at[srcv.at[base + j]], bset[j], gsem)

        def fire_scatters(bset, base):
            for j in range(grp):
                pltpu.async_copy(bset[j], acc.at[dstv.at[base + j]], ssem,
                                 add=True)
                if with_cnt:
                    pltpu.async_copy(obuf, cacc.at[dstv.at[base + j]], osem,
                                     add=True)

        def drain_scatters(bset):
            for j in range(grp):
                pltpu.make_async_copy(bset[j], acc.at[dstv.at[0]],
                                      ssem).wait()

        fire_gathers(bufs[0], 0)

        def pair(q, carry):
            a = 2 * q * grp
            wait_gathers(bufs[0])

            @pl.when(q > 0)
            def _d1():
                drain_scatters(bufs[1])

            fire_gathers(bufs[1], a + grp)
            fire_scatters(bufs[0], a)
            wait_gathers(bufs[1])

            @pl.when(q + 1 < npair)
            def _d0():
                drain_scatters(bufs[0])
                fire_gathers(bufs[0], a + 2 * grp)

            fire_scatters(bufs[1], a + grp)
            return carry

        lax.fori_loop(0, npair, pair, 0)
        drain_scatters(bufs[0])
        drain_scatters(bufs[1])
        if with_cnt:
            def odrain(k, carry):
                pltpu.make_async_copy(obuf, cacc.at[dstv.at[0]], osem).wait()
                return carry

            lax.fori_loop(0, nch, odrain, 0)
        plsc.subcore_barrier()

        def cout(k, carry):
            pltpu.sync_copy(acc.at[pl.ds(sid * rpt + k * _CH, _CH)],
                            bufs[0][0])
            pltpu.async_copy(bufs[0][0],
                             out.at[cid, pl.ds(sid * rpt + k * _CH, _CH)],
                             gsem).wait()
            if with_cnt:
                pltpu.sync_copy(cacc.at[pl.ds(sid * rpt + k * _CH, _CH)],
                                bufs[1][0])
                pltpu.async_copy(
                    bufs[1][0],
                    cout_hbm.at[cid, pl.ds(sid * rpt + k * _CH, _CH)],
                    gsem).wait()
            return carry

        lax.fori_loop(0, rpt // _CH, cout, 0)

    return seg


def kernel(x, edge_index, Wl1, bl1, Wr1, g1, b1, Wl2, bl2, Wr2, g2, b2,
           Wl3, bl3, Wr3):
    n, d_in = x.shape
    e = edge_index.shape[1]
    d_h = Wl1.shape[0]
    d_out = Wl3.shape[0]
    f32 = jnp.float32

    pk = 128 // d_h
    n_pad = _pad_up(n + 1, 4 * _NSUB * _CH // d_h)
    e_pad = _pad_up(e, _NWORK * _CH * 10)
    nch = e_pad // (_NWORK * _CH)

    npk = n // pk
    src = edge_index[0]
    dst = edge_index[1]
    src = pk * (src % npk) + src // npk
    dst = pk * (dst % npk) + dst // npk
    pad = e_pad - e
    if pad:
        ar = jnp.arange(pad, dtype=jnp.int32)
        src = jnp.concatenate([src, ar % n])
        dst = jnp.concatenate([dst, n + ar % (n_pad - n)])
    src2d = src.reshape(e_pad // _CH, _CH)
    dst2d = dst.reshape(e_pad // _CH, _CH)

    eye = jnp.eye(pk, dtype=f32)
    wl1b = jnp.kron(eye, Wl1.T)
    wr1b = jnp.kron(eye, Wr1.T)
    wl2b = jnp.kron(eye, Wl2.T)
    wr2b = jnp.kron(eye, Wr2.T)
    w3p = jnp.zeros((d_h, d_h), f32).at[:, :d_out].set(Wl3.T)
    w3rp = jnp.zeros((d_h, d_h), f32).at[:, :d_out].set(Wr3.T)
    wl3b = jnp.kron(eye, w3p)
    wr3b = jnp.kron(eye, w3rp)
    mavg = jnp.kron(eye, jnp.full((d_h, d_h), 1.0 / d_h, f32))
    bl1p = jnp.tile(bl1, pk).reshape(1, 128)
    g1p = jnp.tile(g1, pk).reshape(1, 128)
    b1p = jnp.tile(b1, pk).reshape(1, 128)
    bl2p = jnp.tile(bl2, pk).reshape(1, 128)
    g2p = jnp.tile(g2, pk).reshape(1, 128)
    b2p = jnp.tile(b2, pk).reshape(1, 128)
    bl3p = jnp.tile(jnp.zeros((d_h,), f32).at[:d_out].set(bl3),
                    pk).reshape(1, 128)

    rbp = npk
    grid = (1,)
    rs = lambda: pl.BlockSpec((npk, 128), lambda i: (0, 0))
    a3 = lambda: pl.BlockSpec((_NCORE, n_pad // pk, 128), lambda i: (0, 0, 0))
    fs = lambda r: pl.BlockSpec((r, 128), lambda i: (0, 0))

    def tc1(x_ref, wl_ref, wr_ref, tab_ref, r_ref):
        xv = x_ref[:]
        xb = jnp.concatenate(
            [xv[k * npk:(k + 1) * npk] for k in range(pk)], axis=1)
        tab_ref[:] = jnp.dot(xb, wl_ref[:], preferred_element_type=f32)
        r_ref[:] = jnp.dot(xb, wr_ref[:], preferred_element_type=f32)

    table1p, r1p = pl.pallas_call(
        tc1,
        grid=grid,
        in_specs=[pl.BlockSpec((n, d_in), lambda i: (0, 0)),
                  pl.BlockSpec((pk * d_in, 128), lambda i: (0, 0)),
                  pl.BlockSpec((pk * d_in, 128), lambda i: (0, 0))],
        out_specs=[rs(), rs()],
        out_shape=[jax.ShapeDtypeStruct((npk, 128), f32),
                   jax.ShapeDtypeStruct((npk, 128), f32)],
    )(x, wl1b, wr1b)

    seg1 = _make_segsum(n_pad, d_h, nch, with_cnt=True)
    acc1, cnt1 = seg1(table1p.reshape(n, d_h), src2d, dst2d)
    acc1p = acc1.reshape(_NCORE, n_pad // pk, 128)
    cnt1p = cnt1.reshape(_NCORE, n_pad // pk, 128)

    def tc2(a_ref, c_ref, r_ref, bl_ref, g_ref, b_ref,
            mavg_ref, wl_ref, wr_ref, tab_ref, rn_ref, inv_ref):
        s = a_ref[0, :npk, :] + a_ref[1, :npk, :]
        inv = 1.0 / jnp.clip(c_ref[0, :npk, :] + c_ref[1, :npk, :], 1.0, None)
        pre = s * inv + bl_ref[:] + r_ref[:]
        mu = jnp.dot(pre, mavg_ref[:], preferred_element_type=f32)
        d = pre - mu
        var = jnp.dot(d * d, mavg_ref[:], preferred_element_type=f32)
        h = d / jnp.sqrt(var + 1e-5) * g_ref[:] + b_ref[:]
        h = jnp.maximum(h, 0.0)
        tab_ref[:] = jnp.dot(h, wl_ref[:], preferred_element_type=f32)
        rn_ref[:] = jnp.dot(h, wr_ref[:], preferred_element_type=f32)
        inv_ref[:] = inv

    table2p, r2p, invp = pl.pallas_call(
        tc2,
        grid=grid,
        in_specs=[a3(), a3(), rs(),
                  fs(1), fs(1), fs(1), fs(128), fs(128), fs(128)],
        out_specs=[rs(), rs(), rs()],
        out_shape=[jax.ShapeDtypeStruct((npk, 128), f32),
                   jax.ShapeDtypeStruct((npk, 128), f32),
                   jax.ShapeDtypeStruct((npk, 128), f32)],
    )(acc1p, cnt1p, r1p, bl1p, g1p, b1p, mavg, wl2b, wr2b)

    seg2 = _make_segsum(n_pad, d_h, nch, with_cnt=False)
    acc2 = seg2(table2p.reshape(n, d_h), src2d, dst2d)
    acc2p = acc2.reshape(_NCORE, n_pad // pk, 128)

    def tc3(a_ref, r_ref, inv_ref, bl_ref, g_ref, b_ref, mavg_ref,
            wl_ref, wr_ref, tab_ref, rn_ref):
        s = a_ref[0, :npk, :] + a_ref[1, :npk, :]
        pre = s * inv_ref[:] + bl_ref[:] + r_ref[:]
        mu = jnp.dot(pre, mavg_ref[:], preferred_element_type=f32)
        d = pre - mu
        var = jnp.dot(d * d, mavg_ref[:], preferred_element_type=f32)
        h = d / jnp.sqrt(var + 1e-5) * g_ref[:] + b_ref[:]
        h = jnp.maximum(h, 0.0)
        tab_ref[:] = jnp.dot(h, wl_ref[:], preferred_element_type=f32)
        rn_ref[:] = jnp.dot(h, wr_ref[:], preferred_element_type=f32)

    table3p, r3p = pl.pallas_call(
        tc3,
        grid=grid,
        in_specs=[a3(), rs(), rs(),
                  fs(1), fs(1), fs(1), fs(128), fs(128), fs(128)],
        out_specs=[rs(), rs()],
        out_shape=[jax.ShapeDtypeStruct((npk, 128), f32),
                   jax.ShapeDtypeStruct((npk, 128), f32)],
    )(acc2p, r2p, invp, bl2p, g2p, b2p, mavg, wl3b, wr3b)

    seg3 = _make_segsum(n_pad, d_h, nch, with_cnt=False)
    acc3 = seg3(table3p.reshape(n, d_h), src2d, dst2d)
    acc3p = acc3.reshape(_NCORE, n_pad // pk, 128)

    def tc4(a_ref, r_ref, inv_ref, bl_ref, out_ref):
        s = a_ref[0, :npk, :] + a_ref[1, :npk, :]
        out_ref[:] = s * inv_ref[:] + bl_ref[:] + r_ref[:]

    outp = pl.pallas_call(
        tc4,
        grid=grid,
        in_specs=[a3(), rs(), rs(), fs(1)],
        out_specs=rs(),
        out_shape=jax.ShapeDtypeStruct((npk, 128), f32),
    )(acc3p, r3p, invp, bl3p)

    return (outp.reshape(npk, pk, d_h)[:, :, :d_out]
            .transpose(1, 0, 2).reshape(n, d_out))


# speedup vs baseline: 1.1362x; 1.1362x over previous
import functools

import jax
import jax.numpy as jnp
from jax import lax
from jax.experimental import pallas as pl
from jax.experimental.pallas import tpu as pltpu
from jax.experimental.pallas import tpu_sc as plsc

_CH = 128
_NCORE = 2
_NSUB = 16
_NWORK = _NCORE * _NSUB


def _pad_up(v, m):
    return (v + m - 1) // m * m


def _make_segsum(n_pad, w, nch, with_cnt):
    rpt = n_pad // _NSUB
    mesh = plsc.VectorSubcoreMesh(core_axis_name="c", subcore_axis_name="s")

    grp = 5 if with_cnt else 10
    assert nch % (2 * grp) == 0
    ngrp = nch // grp

    out_shape = jax.ShapeDtypeStruct((_NCORE, n_pad, w), jnp.float32)
    scratch = [
        pltpu.VMEM((nch, _CH), jnp.int32),
        pltpu.VMEM((nch, _CH), jnp.int32),
        *[pltpu.VMEM((_CH, w), jnp.float32) for _ in range(2 * grp)],
        pltpu.VMEM_SHARED((n_pad, w), jnp.float32),
        pltpu.SemaphoreType.DMA,
        pltpu.SemaphoreType.DMA,
    ]
    if with_cnt:
        scratch += [
            pltpu.VMEM((_CH, w), jnp.float32),
            pltpu.VMEM_SHARED((n_pad, w), jnp.float32),
            pltpu.SemaphoreType.DMA,
        ]

    @functools.partial(
        pl.kernel,
        out_type=[out_shape, out_shape] if with_cnt else out_shape,
        mesh=mesh,
        compiler_params=pltpu.CompilerParams(use_tc_tiling_on_sc=False),
        scratch_types=scratch,
    )
    def seg(table, src2d, dst2d, *rest):
        if with_cnt:
            out, cout_hbm = rest[0], rest[1]
            rest = rest[2:]
        else:
            out = rest[0]
            rest = rest[1:]
        srcv, dstv = rest[0], rest[1]
        bufs = (rest[2:2 + grp], rest[2 + grp:2 + 2 * grp])
        acc = rest[2 + 2 * grp]
        gsem, ssem = rest[3 + 2 * grp], rest[4 + 2 * grp]
        if with_cnt:
            obuf, cacc, osem = (rest[5 + 2 * grp], rest[6 + 2 * grp],
                                rest[7 + 2 * grp])
        cid = lax.axis_index("c")
        sid = lax.axis_index("s")
        wid = cid * _NSUB + sid

        pltpu.sync_copy(src2d.at[pl.ds(wid * nch, nch)], srcv)
        pltpu.sync_copy(dst2d.at[pl.ds(wid * nch, nch)], dstv)

        zv = jnp.zeros((16,), jnp.float32)

        def zrow(i, carry):
            for j in range(w // 16):
                bufs[0][0][i, pl.ds(j * 16, 16)] = zv
            return carry

        lax.fori_loop(0, _CH, zrow, 0)

        def zcopy(k, carry):
            pltpu.sync_copy(bufs[0][0],
                            acc.at[pl.ds(sid * rpt + k * _CH, _CH)])
            if with_cnt:
                pltpu.sync_copy(bufs[0][0],
                                cacc.at[pl.ds(sid * rpt + k * _CH, _CH)])
            return carry

        lax.fori_loop(0, rpt // _CH, zcopy, 0)

        if with_cnt:
            ov = jnp.ones((16,), jnp.float32)

            def orow(i, carry):
                for j in range(w // 16):
                    obuf[i, pl.ds(j * 16, 16)] = ov
                return carry

            lax.fori_loop(0, _CH, orow, 0)

        plsc.subcore_barrier()

        npair = ngrp // 2

        def wait_gathers(bset):
            for j in range(grp):
                pltpu.make_async_copy(table.at[srcv.at[0]], bset[j],
                                      gsem).wait()

        def fire_gathers(bset, base):
            for j in range(grp):
                pltpu.async_copy(table.at[srcv.at[base + j]], bset[j], gsem)

        def fire_scatters(bset, base):
            for j in range(grp):
                pltpu.async_copy(bset[j], acc.at[dstv.at[base + j]], ssem,
                                 add=True)
                if with_cnt:
                    pltpu.async_copy(obuf, cacc.at[dstv.at[base + j]], osem,
                                     add=True)

        def drain_scatters(bset):
            for j in range(grp):
                pltpu.make_async_copy(bset[j], acc.at[dstv.at[0]],
                                      ssem).wait()

        fire_gathers(bufs[0], 0)

        def pair(q, carry):
            a = 2 * q * grp
            wait_gathers(bufs[0])

            @pl.when(q > 0)
            def _d1():
                drain_scatters(bufs[1])

            fire_gathers(bufs[1], a + grp)
            fire_scatters(bufs[0], a)
            wait_gathers(bufs[1])

            @pl.when(q + 1 < npair)
            def _d0():
                drain_scatters(bufs[0])
                fire_gathers(bufs[0], a + 2 * grp)

            fire_scatters(bufs[1], a + grp)
            return carry

        lax.fori_loop(0, npair, pair, 0)
        drain_scatters(bufs[0])
        drain_scatters(bufs[1])
        if with_cnt:
            def odrain(k, carry):
                pltpu.make_async_copy(obuf, cacc.at[dstv.at[0]], osem).wait()
                return carry

            lax.fori_loop(0, nch, odrain, 0)
        plsc.subcore_barrier()

        def cout(k, carry):
            pltpu.sync_copy(acc.at[pl.ds(sid * rpt + k * _CH, _CH)],
                            bufs[0][0])
            pltpu.async_copy(bufs[0][0],
                             out.at[cid, pl.ds(sid * rpt + k * _CH, _CH)],
                             gsem).wait()
            if with_cnt:
                pltpu.sync_copy(cacc.at[pl.ds(sid * rpt + k * _CH, _CH)],
                                bufs[1][0])
                pltpu.async_copy(
                    bufs[1][0],
                    cout_hbm.at[cid, pl.ds(sid * rpt + k * _CH, _CH)],
                    gsem).wait()
            return carry

        lax.fori_loop(0, rpt // _CH, cout, 0)

    return seg


def kernel(x, edge_index, Wl1, bl1, Wr1, g1, b1, Wl2, bl2, Wr2, g2, b2,
           Wl3, bl3, Wr3):
    n, d_in = x.shape
    e = edge_index.shape[1]
    d_h = Wl1.shape[0]
    d_out = Wl3.shape[0]
    f32 = jnp.float32

    pk = 128 // d_h
    n_pad = _pad_up(n + 1, 4 * _NSUB * _CH // d_h)
    e_pad = _pad_up(e, _NWORK * _CH * 10)
    nch = e_pad // (_NWORK * _CH)

    src = edge_index[0]
    dst = edge_index[1]
    pad = e_pad - e
    if pad:
        ar = jnp.arange(pad, dtype=jnp.int32)
        src = jnp.concatenate([src, ar % n])
        dst = jnp.concatenate([dst, n + ar % (n_pad - n)])
    src2d = src.reshape(e_pad // _CH, _CH)
    dst2d = dst.reshape(e_pad // _CH, _CH)

    eye = jnp.eye(pk, dtype=f32)
    wl1b = jnp.kron(eye, Wl1.T)
    wr1b = jnp.kron(eye, Wr1.T)
    wl2b = jnp.kron(eye, Wl2.T)
    wr2b = jnp.kron(eye, Wr2.T)
    w3p = jnp.zeros((d_h, d_h), f32).at[:, :d_out].set(Wl3.T)
    w3rp = jnp.zeros((d_h, d_h), f32).at[:, :d_out].set(Wr3.T)
    wl3b = jnp.kron(eye, w3p)
    wr3b = jnp.kron(eye, w3rp)
    mavg = jnp.kron(eye, jnp.full((d_h, d_h), 1.0 / d_h, f32))
    bl1p = jnp.tile(bl1, pk).reshape(1, 128)
    g1p = jnp.tile(g1, pk).reshape(1, 128)
    b1p = jnp.tile(b1, pk).reshape(1, 128)
    bl2p = jnp.tile(bl2, pk).reshape(1, 128)
    g2p = jnp.tile(g2, pk).reshape(1, 128)
    b2p = jnp.tile(b2, pk).reshape(1, 128)
    bl3p = jnp.tile(jnp.zeros((d_h,), f32).at[:d_out].set(bl3),
                    pk).reshape(1, 128)

    npk = n // pk
    rbp = npk
    grid = (1,)
    rs = lambda: pl.BlockSpec((npk, 128), lambda i: (0, 0))
    a3 = lambda: pl.BlockSpec((_NCORE, n_pad // pk, 128), lambda i: (0, 0, 0))
    fs = lambda r: pl.BlockSpec((r, 128), lambda i: (0, 0))

    x4 = x.reshape(npk, pk * d_in)

    def tc1(x_ref, wl_ref, wr_ref, tab_ref, r_ref):
        xb = x_ref[:]
        tab_ref[:] = jnp.dot(xb, wl_ref[:], preferred_element_type=f32)
        r_ref[:] = jnp.dot(xb, wr_ref[:], preferred_element_type=f32)

    table1p, r1p = pl.pallas_call(
        tc1,
        grid=grid,
        in_specs=[pl.BlockSpec((npk, pk * d_in), lambda i: (0, 0)),
                  pl.BlockSpec((pk * d_in, 128), lambda i: (0, 0)),
                  pl.BlockSpec((pk * d_in, 128), lambda i: (0, 0))],
        out_specs=[rs(), rs()],
        out_shape=[jax.ShapeDtypeStruct((npk, 128), f32),
                   jax.ShapeDtypeStruct((npk, 128), f32)],
    )(x4, wl1b, wr1b)

    seg1 = _make_segsum(n_pad, d_h, nch, with_cnt=True)
    acc1, cnt1 = seg1(table1p.reshape(n, d_h), src2d, dst2d)
    acc1p = acc1.reshape(_NCORE, n_pad // pk, 128)
    cnt1p = cnt1.reshape(_NCORE, n_pad // pk, 128)

    def tc2(a_ref, c_ref, r_ref, bl_ref, g_ref, b_ref,
            mavg_ref, wl_ref, wr_ref, tab_ref, rn_ref, inv_ref):
        s = a_ref[0, :npk, :] + a_ref[1, :npk, :]
        inv = 1.0 / jnp.clip(c_ref[0, :npk, :] + c_ref[1, :npk, :], 1.0, None)
        pre = s * inv + bl_ref[:] + r_ref[:]
        mu = jnp.dot(pre, mavg_ref[:], preferred_element_type=f32)
        d = pre - mu
        var = jnp.dot(d * d, mavg_ref[:], preferred_element_type=f32)
        h = d / jnp.sqrt(var + 1e-5) * g_ref[:] + b_ref[:]
        h = jnp.maximum(h, 0.0)
        tab_ref[:] = jnp.dot(h, wl_ref[:], preferred_element_type=f32)
        rn_ref[:] = jnp.dot(h, wr_ref[:], preferred_element_type=f32)
        inv_ref[:] = inv

    table2p, r2p, invp = pl.pallas_call(
        tc2,
        grid=grid,
        in_specs=[a3(), a3(), rs(),
                  fs(1), fs(1), fs(1), fs(128), fs(128), fs(128)],
        out_specs=[rs(), rs(), rs()],
        out_shape=[jax.ShapeDtypeStruct((npk, 128), f32),
                   jax.ShapeDtypeStruct((npk, 128), f32),
                   jax.ShapeDtypeStruct((npk, 128), f32)],
    )(acc1p, cnt1p, r1p, bl1p, g1p, b1p, mavg, wl2b, wr2b)

    seg2 = _make_segsum(n_pad, d_h, nch, with_cnt=False)
    acc2 = seg2(table2p.reshape(n, d_h), src2d, dst2d)
    acc2p = acc2.reshape(_NCORE, n_pad // pk, 128)

    def tc3(a_ref, r_ref, inv_ref, bl_ref, g_ref, b_ref, mavg_ref,
            wl_ref, wr_ref, tab_ref, rn_ref):
        s = a_ref[0, :npk, :] + a_ref[1, :npk, :]
        pre = s * inv_ref[:] + bl_ref[:] + r_ref[:]
        mu = jnp.dot(pre, mavg_ref[:], preferred_element_type=f32)
        d = pre - mu
        var = jnp.dot(d * d, mavg_ref[:], preferred_element_type=f32)
        h = d / jnp.sqrt(var + 1e-5) * g_ref[:] + b_ref[:]
        h = jnp.maximum(h, 0.0)
        tab_ref[:] = jnp.dot(h, wl_ref[:], preferred_element_type=f32)
        rn_ref[:] = jnp.dot(h, wr_ref[:], preferred_element_type=f32)

    table3p, r3p = pl.pallas_call(
        tc3,
        grid=grid,
        in_specs=[a3(), rs(), rs(),
                  fs(1), fs(1), fs(1), fs(128), fs(128), fs(128)],
        out_specs=[rs(), rs()],
        out_shape=[jax.ShapeDtypeStruct((npk, 128), f32),
                   jax.ShapeDtypeStruct((npk, 128), f32)],
    )(acc2p, r2p, invp, bl2p, g2p, b2p, mavg, wl3b, wr3b)

    seg3 = _make_segsum(n_pad, d_h, nch, with_cnt=False)
    acc3 = seg3(table3p.reshape(n, d_h), src2d, dst2d)
    acc3p = acc3.reshape(_NCORE, n_pad // pk, 128)

    def tc4(a_ref, r_ref, inv_ref, bl_ref, out_ref):
        s = a_ref[0, :npk, :] + a_ref[1, :npk, :]
        out_ref[:] = s * inv_ref[:] + bl_ref[:] + r_ref[:]

    outp = pl.pallas_call(
        tc4,
        grid=grid,
        in_specs=[a3(), rs(), rs(), fs(1)],
        out_specs=rs(),
        out_shape=jax.ShapeDtypeStruct((npk, 128), f32),
    )(acc3p, r3p, invp, bl3p)

    return outp.reshape(n, d_h)[:, :d_out]
